# baseline (speedup 1.0000x reference)
import jax
import jax.numpy as jnp
from jax import lax
from jax.experimental import pallas as pl
from jax.experimental.pallas import tpu as pltpu
from jax.experimental.pallas import tpu_sc as plsc

VOCAB = 1000000
D = 64
ROWS = 16384
COLS = 50
NC = 2
NS = 16
NW = NC * NS
ROWS_PER_W = ROWS // NW
GROUP = 8
GROUPS = ROWS_PER_W // GROUP

VB = 8192
NB = (VOCAB + VB - 1) // VB


def _wt_body(wt_ref, out_ref, tmp_ref):
    tmp_ref[...] = wt_ref[...].T
    a = tmp_ref[pl.Slice(0, VB // 2, 2), :]
    b = tmp_ref[pl.Slice(1, VB // 2, 2), :]
    out_ref[...] = jnp.concatenate([a, b], axis=1)


def _qt_body(in_ref, out_ref):
    t = in_ref[...].reshape(128, COLS, D).transpose(1, 2, 0)
    out_ref[...] = t.reshape(COLS, 8, 1, 8, 128)


def _fire(table_hbm, idx_v, rows_buf, sem, g):
    for j in range(GROUP):
        pltpu.async_copy(
            table_hbm.at[idx_v.at[g * GROUP + j]],
            rows_buf.at[j],
            sem,
        )


def _drain(table_hbm, idx_v, rows_buf, sem):
    for j in range(GROUP):
        pltpu.make_async_copy(
            table_hbm.at[idx_v.at[j]],
            rows_buf.at[j],
            sem,
        ).wait()


def _emb_body(x_hbm, table_hbm, out_hbm, idx_v, rows0, rows1, sem0, sem1):
    wid = lax.axis_index("s") * NC + lax.axis_index("c")
    row_base = wid * ROWS_PER_W
    pltpu.sync_copy(x_hbm.at[pl.ds(row_base, ROWS_PER_W)], idx_v)

    def store(rows_buf, g):
        pltpu.sync_copy(rows_buf, out_hbm.at[pl.ds(row_base + g * GROUP, GROUP)])

    _fire(table_hbm, idx_v, rows0, sem0, 0)

    def pair_body(i, _):
        g = 2 * i
        _drain(table_hbm, idx_v, rows0, sem0)
        _fire(table_hbm, idx_v, rows1, sem1, g + 1)
        store(rows0, g)
        _drain(table_hbm, idx_v, rows1, sem1)
        _fire(table_hbm, idx_v, rows0, sem0, g + 2)
        store(rows1, g + 1)
        return ()

    lax.fori_loop(0, GROUPS // 2 - 1, pair_body, (), unroll=False)

    g = GROUPS - 2
    _drain(table_hbm, idx_v, rows0, sem0)
    _fire(table_hbm, idx_v, rows1, sem1, g + 1)
    store(rows0, g)
    _drain(table_hbm, idx_v, rows1, sem1)
    store(rows1, g + 1)


@jax.jit
def _emb_call(x, weight):
    w2 = pl.pallas_call(
        _wt_body,
        grid=(NB,),
        in_specs=[pl.BlockSpec((D, VB), lambda i: (0, i))],
        out_specs=pl.BlockSpec((VB // 2, 128), lambda i: (i, 0)),
        out_shape=jax.ShapeDtypeStruct((VOCAB // 2, 128), jnp.float32),
        scratch_shapes=[pltpu.VMEM((VB, D), jnp.float32)],
    )(weight.T)
    wlin = w2.reshape(VOCAB, D)

    mesh = plsc.VectorSubcoreMesh(core_axis_name="c", subcore_axis_name="s")
    out3 = pl.kernel(
        _emb_body,
        out_type=jax.ShapeDtypeStruct((ROWS, COLS, D), jnp.float32),
        mesh=mesh,
        scratch_types=[
            pltpu.VMEM((ROWS_PER_W, COLS), jnp.int32),
            pltpu.VMEM((GROUP, COLS, D), jnp.float32),
            pltpu.VMEM((GROUP, COLS, D), jnp.float32),
            pltpu.SemaphoreType.DMA,
            pltpu.SemaphoreType.DMA,
        ],
        compiler_params=pltpu.CompilerParams(use_tc_tiling_on_sc=False),
    )(x, wlin)

    q5 = pl.pallas_call(
        _qt_body,
        grid=(ROWS // 128,),
        in_specs=[pl.BlockSpec((128, COLS * D), lambda i: (i, 0))],
        out_specs=pl.BlockSpec((COLS, 8, 1, 8, 128), lambda i: (0, 0, i, 0, 0)),
        out_shape=jax.ShapeDtypeStruct((COLS, 8, ROWS // 128, 8, 128), jnp.float32),
    )(out3.reshape(ROWS, COLS * D))

    return q5.transpose(2, 4, 0, 1, 3).reshape(ROWS, COLS, D)


def kernel(x, weight):
    return _emb_call(x.astype(jnp.int32), weight)


# speedup vs baseline: 1.2074x; 1.2074x over previous
import jax
import jax.numpy as jnp
from jax import lax
from jax.experimental import pallas as pl
from jax.experimental.pallas import tpu as pltpu
from jax.experimental.pallas import tpu_sc as plsc

VOCAB = 1000000
D = 64
ROWS = 16384
COLS = 50
NC = 2
NS = 16
NW = NC * NS
ROWS_PER_W = ROWS // NW
GROUP = 16
GROUPS = ROWS_PER_W // GROUP


def _fire(table_hbm, idx_v, rows_buf, sem, g):
    for j in range(GROUP):
        pltpu.async_copy(
            table_hbm.at[idx_v.at[g * GROUP + j]],
            rows_buf.at[j],
            sem,
        )


def _drain(table_hbm, idx_v, rows_buf, sem):
    for j in range(GROUP):
        pltpu.make_async_copy(
            table_hbm.at[idx_v.at[j]],
            rows_buf.at[j],
            sem,
        ).wait()


def _emb_body(x_hbm, table_hbm, out_hbm, idx_v, rows0, rows1, sem0, sem1):
    wid = lax.axis_index("s") * NC + lax.axis_index("c")
    row_base = wid * ROWS_PER_W
    pltpu.sync_copy(x_hbm.at[pl.ds(row_base, ROWS_PER_W)], idx_v)

    def store(rows_buf, g):
        pltpu.sync_copy(rows_buf, out_hbm.at[pl.ds(row_base + g * GROUP, GROUP)])

    _fire(table_hbm, idx_v, rows0, sem0, 0)

    def pair_body(i, _):
        g = 2 * i
        _drain(table_hbm, idx_v, rows0, sem0)
        _fire(table_hbm, idx_v, rows1, sem1, g + 1)
        store(rows0, g)
        _drain(table_hbm, idx_v, rows1, sem1)
        _fire(table_hbm, idx_v, rows0, sem0, g + 2)
        store(rows1, g + 1)
        return ()

    lax.fori_loop(0, GROUPS // 2 - 1, pair_body, (), unroll=False)

    g = GROUPS - 2
    _drain(table_hbm, idx_v, rows0, sem0)
    _fire(table_hbm, idx_v, rows1, sem1, g + 1)
    store(rows0, g)
    _drain(table_hbm, idx_v, rows1, sem1)
    store(rows1, g + 1)


@jax.jit
def _emb_call(x, weight):
    mesh = plsc.VectorSubcoreMesh(core_axis_name="c", subcore_axis_name="s")
    return pl.kernel(
        _emb_body,
        out_type=jax.ShapeDtypeStruct((ROWS, COLS, D), jnp.float32),
        mesh=mesh,
        scratch_types=[
            pltpu.VMEM((ROWS_PER_W, COLS), jnp.int32),
            pltpu.VMEM((GROUP, COLS, D), jnp.float32),
            pltpu.VMEM((GROUP, COLS, D), jnp.float32),
            pltpu.SemaphoreType.DMA,
            pltpu.SemaphoreType.DMA,
        ],
        compiler_params=pltpu.CompilerParams(use_tc_tiling_on_sc=False),
    )(x, weight)


def kernel(x, weight):
    return _emb_call(x.astype(jnp.int32), weight)
